# baseline (speedup 1.0000x reference)
import functools

import jax
import jax.numpy as jnp
from jax import lax
from jax.experimental import pallas as pl
from jax.experimental.pallas import tpu as pltpu

H = 2048
I = 8192
E = 64
ES = 8
K = 8

_DN_T = (((1,), (1,)), ((), ()))


def _router_kernel(x_ref, rxw_ref, ryw_ref, ue_ref, de_ref, xb_ref,
                   experts_ref, lx_s, ly_s, s_s, *, n_tiles, tn):
    n = pl.program_id(0)

    @pl.when(n < n_tiles)
    def _():
        x = x_ref[...]
        rows = pl.ds(n * tn, tn)
        lx_s[rows, :] = lax.dot_general(x, rxw_ref[...], _DN_T,
                                        preferred_element_type=jnp.float32)
        ly_s[rows, :] = lax.dot_general(x, ryw_ref[...], _DN_T,
                                        preferred_element_type=jnp.float32)
        s_s[rows, :] = lax.dot_general(x, ue_ref[...], _DN_T,
                                       preferred_element_type=jnp.float32)
        xb_ref[...] = x.astype(jnp.bfloat16)

    @pl.when(n == n_tiles)
    def _():
        contrib = _routing_math(lx_s[...], ly_s[...], s_s[...])
        experts_ref[...] = lax.dot_general(
            contrib.astype(jnp.bfloat16), de_ref[...].astype(jnp.bfloat16),
            (((1,), (0,)), ((), ())),
            preferred_element_type=jnp.float32).astype(jnp.bfloat16)


def _routing_math(lx, ly, s):
    def bn(z):
        m = jnp.mean(z, axis=0, keepdims=True)
        v = jnp.mean((z - m) * (z - m), axis=0, keepdims=True)
        return (z - m) * lax.rsqrt(v + 1e-5)

    def lsm(z):
        zm = z - jnp.max(z, axis=1, keepdims=True)
        return zm - jnp.log(jnp.sum(jnp.exp(zm), axis=1, keepdims=True))

    lpx = lsm(bn(lx))
    lpy = lsm(bn(ly))

    r = lax.broadcasted_iota(jnp.int32, (ES, E), 0)
    c = lax.broadcasted_iota(jnp.int32, (ES, E), 1)
    sel_x = (r == c // ES).astype(jnp.float32)
    sel_y = (r == c % ES).astype(jnp.float32)
    dn = (((1,), (0,)), ((), ()))
    combined = (lax.dot_general(lpx, sel_x, dn, preferred_element_type=jnp.float32)
                + lax.dot_general(lpy, sel_y, dn, preferred_element_type=jnp.float32))

    n = combined.shape[0]
    colf = lax.broadcasted_iota(jnp.int32, (n, E), 1).astype(jnp.float32)
    cur = combined
    for _ in range(K):
        m = jnp.max(cur, axis=1, keepdims=True)
        first = jnp.min(jnp.where(cur == m, colf, float(E)),
                        axis=1, keepdims=True)
        cur = jnp.where(colf == first, -jnp.inf, cur)

    w = jnp.where(jnp.isneginf(cur), jnp.exp(combined), 0.0)
    return w * (s * jax.nn.sigmoid(s))


_N_TILES_R = 4
_TN_R = 1024


def _mlp_kernel(x_ref, gw_ref, uw_ref, dw_ref, experts_ref, out_ref):
    i = pl.program_id(1)

    @pl.when(i == 0)
    def _():
        out_ref[...] = experts_ref[...].astype(jnp.float32)

    x = x_ref[...]
    gw = gw_ref[...].astype(jnp.bfloat16)
    uw = uw_ref[...].astype(jnp.bfloat16)
    dw = dw_ref[...].astype(jnp.bfloat16)
    g = lax.dot_general(x, gw, _DN_T, preferred_element_type=jnp.float32)
    u = lax.dot_general(x, uw, _DN_T, preferred_element_type=jnp.float32)
    h = (g * jax.nn.sigmoid(g) * u).astype(jnp.bfloat16)
    out_ref[...] += lax.dot_general(h, dw, _DN_T,
                                    preferred_element_type=jnp.float32)


def kernel(hidden_states, gate_w, up_w, down_w, router_x_w, router_y_w,
           up_embed, down_embed):
    bsz, seq, _ = hidden_states.shape
    x = hidden_states.reshape(-1, H)
    N = x.shape[0]

    nt = N // _TN_R
    last = nt - 1
    xb, experts = pl.pallas_call(
        functools.partial(_router_kernel, n_tiles=nt, tn=_TN_R),
        grid=(nt + 1,),
        in_specs=[
            pl.BlockSpec((_TN_R, H), lambda n: (jnp.minimum(n, last), 0)),
            pl.BlockSpec((ES, H), lambda n: (0, 0)),
            pl.BlockSpec((ES, H), lambda n: (0, 0)),
            pl.BlockSpec((E, H), lambda n: (0, 0)),
            pl.BlockSpec((E, H), lambda n: (0, 0)),
        ],
        out_specs=[
            pl.BlockSpec((_TN_R, H), lambda n: (jnp.minimum(n, last), 0)),
            pl.BlockSpec((N, H), lambda n: (0, 0)),
        ],
        out_shape=[
            jax.ShapeDtypeStruct((N, H), jnp.bfloat16),
            jax.ShapeDtypeStruct((N, H), jnp.bfloat16),
        ],
        scratch_shapes=[
            pltpu.VMEM((N, ES), jnp.float32),
            pltpu.VMEM((N, ES), jnp.float32),
            pltpu.VMEM((N, E), jnp.float32),
        ],
    )(x, router_x_w, router_y_w, up_embed, down_embed)

    TN = 1024
    TI = 512
    out = pl.pallas_call(
        _mlp_kernel,
        grid=(N // TN, I // TI),
        in_specs=[
            pl.BlockSpec((TN, H), lambda n, i: (n, 0)),
            pl.BlockSpec((TI, H), lambda n, i: (i, 0)),
            pl.BlockSpec((TI, H), lambda n, i: (i, 0)),
            pl.BlockSpec((H, TI), lambda n, i: (0, i)),
            pl.BlockSpec((TN, H), lambda n, i: (n, 0)),
        ],
        out_specs=pl.BlockSpec((TN, H), lambda n, i: (n, 0)),
        out_shape=jax.ShapeDtypeStruct((N, H), jnp.float32),
        compiler_params=pltpu.CompilerParams(
            vmem_limit_bytes=100 * 1024 * 1024),
    )(xb, gate_w, up_w, down_w, experts)

    return out.reshape(bsz, seq, H)


# speedup vs baseline: 1.0081x; 1.0081x over previous
import functools

import jax
import jax.numpy as jnp
from jax import lax
from jax.experimental import pallas as pl
from jax.experimental.pallas import tpu as pltpu

H = 2048
I = 8192
E = 64
ES = 8
K = 8

_DN_T = (((1,), (1,)), ((), ()))


def _router_kernel(x_ref, rxw_ref, ryw_ref, ue_ref, xb_ref,
                   contrib_ref, lx_s, ly_s, s_s, *, n_tiles, tn):
    n = pl.program_id(0)

    @pl.when(n < n_tiles)
    def _():
        x = x_ref[...]
        rows = pl.ds(n * tn, tn)
        lx_s[rows, :] = lax.dot_general(x, rxw_ref[...], _DN_T,
                                        preferred_element_type=jnp.float32)
        ly_s[rows, :] = lax.dot_general(x, ryw_ref[...], _DN_T,
                                        preferred_element_type=jnp.float32)
        s_s[rows, :] = lax.dot_general(x, ue_ref[...], _DN_T,
                                       preferred_element_type=jnp.float32)
        xb_ref[...] = x.astype(jnp.bfloat16)

    @pl.when(n == n_tiles)
    def _():
        contrib_ref[...] = _routing_math(lx_s[...], ly_s[...], s_s[...])


def _routing_math(lx, ly, s):
    def bn(z):
        m = jnp.mean(z, axis=0, keepdims=True)
        v = jnp.mean((z - m) * (z - m), axis=0, keepdims=True)
        return (z - m) * lax.rsqrt(v + 1e-5)

    def lsm(z):
        zm = z - jnp.max(z, axis=1, keepdims=True)
        return zm - jnp.log(jnp.sum(jnp.exp(zm), axis=1, keepdims=True))

    lpx = lsm(bn(lx))
    lpy = lsm(bn(ly))

    r = lax.broadcasted_iota(jnp.int32, (ES, E), 0)
    c = lax.broadcasted_iota(jnp.int32, (ES, E), 1)
    sel_x = (r == c // ES).astype(jnp.float32)
    sel_y = (r == c % ES).astype(jnp.float32)
    dn = (((1,), (0,)), ((), ()))
    combined = (lax.dot_general(lpx, sel_x, dn, preferred_element_type=jnp.float32)
                + lax.dot_general(lpy, sel_y, dn, preferred_element_type=jnp.float32))

    n = combined.shape[0]
    colf = lax.broadcasted_iota(jnp.int32, (n, E), 1).astype(jnp.float32)
    cur = combined
    for _ in range(K):
        m = jnp.max(cur, axis=1, keepdims=True)
        first = jnp.min(jnp.where(cur == m, colf, float(E)),
                        axis=1, keepdims=True)
        cur = jnp.where(colf == first, -jnp.inf, cur)

    w = jnp.where(jnp.isneginf(cur), jnp.exp(combined), 0.0)
    return (w * (s * jax.nn.sigmoid(s))).astype(jnp.bfloat16)


_N_TILES_R = 4
_TN_R = 1024


def _mlp_kernel(x_ref, gw_ref, uw_ref, dw_ref, contrib_ref, de_ref, out_ref):
    i = pl.program_id(1)

    @pl.when(i == 0)
    def _():
        out_ref[...] = lax.dot_general(
            contrib_ref[...], de_ref[...].astype(jnp.bfloat16),
            (((1,), (0,)), ((), ())),
            preferred_element_type=jnp.float32)

    x = x_ref[...]
    gw = gw_ref[...].astype(jnp.bfloat16)
    uw = uw_ref[...].astype(jnp.bfloat16)
    dw = dw_ref[...].astype(jnp.bfloat16)
    g = lax.dot_general(x, gw, _DN_T, preferred_element_type=jnp.float32)
    u = lax.dot_general(x, uw, _DN_T, preferred_element_type=jnp.float32)
    h = (g * jax.nn.sigmoid(g) * u).astype(jnp.bfloat16)
    out_ref[...] += lax.dot_general(h, dw, _DN_T,
                                    preferred_element_type=jnp.float32)


def kernel(hidden_states, gate_w, up_w, down_w, router_x_w, router_y_w,
           up_embed, down_embed):
    bsz, seq, _ = hidden_states.shape
    x = hidden_states.reshape(-1, H)
    N = x.shape[0]

    nt = N // _TN_R
    last = nt - 1
    xb, contrib = pl.pallas_call(
        functools.partial(_router_kernel, n_tiles=nt, tn=_TN_R),
        grid=(nt + 1,),
        in_specs=[
            pl.BlockSpec((_TN_R, H), lambda n: (jnp.minimum(n, last), 0)),
            pl.BlockSpec((ES, H), lambda n: (0, 0)),
            pl.BlockSpec((ES, H), lambda n: (0, 0)),
            pl.BlockSpec((E, H), lambda n: (0, 0)),
        ],
        out_specs=[
            pl.BlockSpec((_TN_R, H), lambda n: (jnp.minimum(n, last), 0)),
            pl.BlockSpec((N, E), lambda n: (0, 0)),
        ],
        out_shape=[
            jax.ShapeDtypeStruct((N, H), jnp.bfloat16),
            jax.ShapeDtypeStruct((N, E), jnp.bfloat16),
        ],
        scratch_shapes=[
            pltpu.VMEM((N, ES), jnp.float32),
            pltpu.VMEM((N, ES), jnp.float32),
            pltpu.VMEM((N, E), jnp.float32),
        ],
    )(x, router_x_w, router_y_w, up_embed)

    TN = 1024
    TI = 512
    out = pl.pallas_call(
        _mlp_kernel,
        grid=(N // TN, I // TI),
        in_specs=[
            pl.BlockSpec((TN, H), lambda n, i: (n, 0)),
            pl.BlockSpec((TI, H), lambda n, i: (i, 0)),
            pl.BlockSpec((TI, H), lambda n, i: (i, 0)),
            pl.BlockSpec((H, TI), lambda n, i: (0, i)),
            pl.BlockSpec((TN, E), lambda n, i: (n, 0)),
            pl.BlockSpec((E, H), lambda n, i: (0, 0)),
        ],
        out_specs=pl.BlockSpec((TN, H), lambda n, i: (n, 0)),
        out_shape=jax.ShapeDtypeStruct((N, H), jnp.float32),
        compiler_params=pltpu.CompilerParams(
            vmem_limit_bytes=63 * 1024 * 1024),
    )(xb, gate_w, up_w, down_w, contrib, down_embed)

    return out.reshape(bsz, seq, H)
